# Initial kernel scaffold; baseline (speedup 1.0000x reference)
import jax
import jax.numpy as jnp
from jax.experimental import pallas as pl


def kernel(index, target, table):
    raise NotImplementedError("write your pallas kernel here")


# speedup vs baseline: 2.4138x; 2.4138x over previous
import dataclasses
import functools

import jax
import jax.numpy as jnp
from jax import lax
from jax.experimental import pallas as pl
from jax.experimental.pallas import tpu as pltpu
from jax.experimental.pallas import tpu_sc as plsc

V = 1000
N = 4096 * 20
L = 16
NW = 32
CPT = N // NW
K = 16
NC = CPT // K
NBUF = 4


def _lse_rows(table):
    def body(t_ref, o_ref):
        t = t_ref[...]
        m = jnp.max(t, axis=1, keepdims=True)
        s = jnp.sum(jnp.exp(t - m), axis=1, keepdims=True)
        o_ref[...] = jnp.log(s) + m

    return pl.pallas_call(
        body,
        out_shape=jax.ShapeDtypeStruct((V, 1), jnp.float32),
    )(table)


def _mean_partials(partials):
    def body(p_ref, o_ref):
        s = jnp.sum(p_ref[...], axis=1, keepdims=True)
        o_ref[...] = jnp.sum(s, axis=0, keepdims=True) * jnp.float32(1.0 / N)

    return pl.pallas_call(
        body,
        out_shape=jax.ShapeDtypeStruct((1, 1), jnp.float32),
    )(partials)


def _sc_gather_and_nll(table, idx_flat, tgt_flat, lse):
    mesh = plsc.VectorSubcoreMesh(core_axis_name="c", subcore_axis_name="s")
    cp = pltpu.CompilerParams()
    if "needs_layout_passes" in pltpu.CompilerParams.__dataclass_fields__:
        cp = dataclasses.replace(cp, needs_layout_passes=False)
    if "use_tc_tiling_on_sc" in pltpu.CompilerParams.__dataclass_fields__:
        cp = dataclasses.replace(cp, use_tc_tiling_on_sc=False)

    @functools.partial(
        pl.kernel,
        compiler_params=cp,
        out_type=(
            jax.ShapeDtypeStruct((N, V), jnp.float32),
            jax.ShapeDtypeStruct((NW, L), jnp.float32),
        ),
        mesh=mesh,
        scratch_types=[
            pltpu.VMEM((CPT,), jnp.int32),
            pltpu.VMEM((CPT,), jnp.int32),
            pltpu.VMEM((V,), jnp.float32),
            pltpu.VMEM((NBUF, K, V), jnp.float32),
            pltpu.VMEM((L,), jnp.float32),
            pltpu.SemaphoreType.DMA,
            [pltpu.SemaphoreType.DMA] * NBUF,
            [pltpu.SemaphoreType.DMA] * NBUF,
        ],
    )
    def k(table_hbm, idx_hbm, tgt_hbm, lse_hbm, out_hbm, part_hbm,
          idx_v, tgt_v, lse_v, rows_v, acc_v, ssem, gsems, osems):
        wid = lax.axis_index("s") * 2 + lax.axis_index("c")
        base = wid * CPT

        pltpu.async_copy(idx_hbm.at[pl.ds(base, CPT)], idx_v, ssem).wait()
        pltpu.async_copy(tgt_hbm.at[pl.ds(base, CPT)], tgt_v, ssem).wait()
        pltpu.async_copy(lse_hbm, lse_v, ssem).wait()
        acc_v[...] = jnp.zeros((L,), jnp.float32)

        def start_gather(c, b):
            pltpu.async_copy(
                table_hbm.at[idx_v.at[pl.ds(c * K, K)]],
                rows_v.at[b], gsems[b])

        def wait_gather(c, b):
            pltpu.make_async_copy(
                table_hbm.at[idx_v.at[pl.ds(c * K, K)]],
                rows_v.at[b], gsems[b]).wait()

        def start_out(c, b):
            pltpu.async_copy(
                rows_v.at[b], out_hbm.at[pl.ds(base + c * K, K)], osems[b])

        def wait_out(c, b):
            pltpu.make_async_copy(
                rows_v.at[b], out_hbm.at[pl.ds(base + c * K, K)], osems[b]
            ).wait()

        start_gather(0, 0)
        start_gather(1, 1)

        row16 = lax.iota(jnp.int32, L)

        @pl.loop(0, NC, step=NBUF)
        def _(g):
            for b in range(NBUF):
                c = g + b
                wait_gather(c, b)
                idx16 = idx_v[pl.ds(c * K, L)]
                tgt16 = tgt_v[pl.ds(c * K, L)]
                vals = plsc.load_gather(rows_v.at[b], [row16, tgt16])
                lsev = plsc.load_gather(lse_v, [idx16])
                acc_v[...] += lsev - vals
                start_out(c, b)
                b2 = (b + 2) % NBUF

                @pl.when(c >= 2)
                def _():
                    wait_out(c - 2, b2)

                @pl.when(c + 2 < NC)
                def _():
                    start_gather(c + 2, b2)

        wait_out(NC - 2, (NC - 2) % NBUF)
        wait_out(NC - 1, (NC - 1) % NBUF)

        pltpu.sync_copy(acc_v, part_hbm.at[wid])

    return k(table, idx_flat, tgt_flat, lse)


def kernel(index, target, table):
    B, T = index.shape
    idx_flat = index.reshape(N).astype(jnp.int32)
    tgt_flat = target.reshape(N).astype(jnp.int32)
    lse = _lse_rows(table).reshape(V)
    logits_flat, partials = _sc_gather_and_nll(table, idx_flat, tgt_flat, lse)
    loss = _mean_partials(partials)[0, 0]
    return logits_flat.reshape(B, T, V), loss
